# baseline (speedup 1.0000x reference)
import functools

import jax
import jax.numpy as jnp
from jax import lax
from jax.experimental import pallas as pl
from jax.experimental.pallas import tpu as pltpu
from jax.experimental.pallas import tpu_sc as plsc

N = 10000
E = 320000
D = 128
H = 128
C = 10

NC = 2
NS = 16
NW = NC * NS

EDGES_PER_W = E // NW
CHUNK = 80
CHUNKS_PER_W = EDGES_PER_W // CHUNK
SB_N = 5
SB_C = CHUNKS_PER_W // SB_N
ROWS_PER_TILE = 632
N_PAD = ROWS_PER_TILE * NS


def _sc_segment_sum_body(h_hbm, src_hbm, dst_hbm, zero_hbm, out_hbm,
                         src_v, dst_v, rows_a, rows_b, agg_sh,
                         sem_ga, sem_gb, sem_sa, sem_sb):
    cid = lax.axis_index("c")
    sid = lax.axis_index("s")
    wid = cid * NS + sid

    pltpu.sync_copy(zero_hbm.at[pl.ds(sid * ROWS_PER_TILE, ROWS_PER_TILE)],
                    agg_sh.at[pl.ds(sid * ROWS_PER_TILE, ROWS_PER_TILE)])
    plsc.subcore_barrier()

    def _gather(i, buf, sem):
        pltpu.async_copy(h_hbm.at[src_v.at[i]], buf, sem)

    def _scatter(i, buf, sem):
        pltpu.async_copy(buf, agg_sh.at[dst_v.at[i]], sem, add=True)

    def _wait(buf, sem):
        pltpu.make_async_copy(h_hbm.at[pl.ds(0, CHUNK)], buf, sem).wait()

    def sb_body(sb, _):
        pltpu.sync_copy(src_hbm.at[wid, sb], src_v)
        pltpu.sync_copy(dst_hbm.at[wid, sb], dst_v)
        _gather(0, rows_a, sem_ga)
        _wait(rows_a, sem_ga)
        _scatter(0, rows_a, sem_sa)
        _gather(1, rows_b, sem_gb)

        def body(k, _):
            i = 2 * k + 1
            _wait(rows_b, sem_gb)
            _wait(rows_a, sem_sa)
            _scatter(i, rows_b, sem_sb)
            _gather(i + 1, rows_a, sem_ga)
            _wait(rows_a, sem_ga)
            _wait(rows_b, sem_sb)
            _scatter(i + 1, rows_a, sem_sa)
            _gather(i + 2, rows_b, sem_gb)
            return 0

        lax.fori_loop(0, (SB_C - 3) // 2, body, 0)
        _wait(rows_b, sem_gb)
        _wait(rows_a, sem_sa)
        _scatter(SB_C - 2, rows_b, sem_sb)
        _gather(SB_C - 1, rows_a, sem_ga)
        _wait(rows_a, sem_ga)
        _wait(rows_b, sem_sb)
        _scatter(SB_C - 1, rows_a, sem_sa)
        _wait(rows_a, sem_sa)
        return 0

    lax.fori_loop(0, SB_N, sb_body, 0)
    plsc.subcore_barrier()

    row0 = sid * ROWS_PER_TILE
    pltpu.sync_copy(agg_sh.at[pl.ds(row0, ROWS_PER_TILE)],
                    out_hbm.at[pl.ds(cid * N_PAD + row0, ROWS_PER_TILE)])


_sc_segment_sum = functools.partial(
    pl.kernel,
    out_type=jax.ShapeDtypeStruct((NC * N_PAD, D), jnp.float32),
    mesh=plsc.VectorSubcoreMesh(core_axis_name="c", subcore_axis_name="s"),
    scratch_types=[
        pltpu.VMEM((SB_C, CHUNK), jnp.int32),
        pltpu.VMEM((SB_C, CHUNK), jnp.int32),
        pltpu.VMEM((CHUNK, D), jnp.float32),
        pltpu.VMEM((CHUNK, D), jnp.float32),
        pltpu.VMEM_SHARED((N_PAD, D), jnp.float32),
        pltpu.SemaphoreType.DMA,
        pltpu.SemaphoreType.DMA,
        pltpu.SemaphoreType.DMA,
        pltpu.SemaphoreType.DMA,
    ],
)(_sc_segment_sum_body)


BN = 2000


def _mm_relu_body(h_ref, p0_ref, p1_ref, w_ref, b_ref, o_ref):
    s = h_ref[...] + p0_ref[...] + p1_ref[...]
    y = jnp.dot(s, w_ref[...], preferred_element_type=jnp.float32) + b_ref[...]
    o_ref[...] = jnp.maximum(y, 0.0)


def _mm_relu(h, p0, p1, w, b):
    return pl.pallas_call(
        _mm_relu_body,
        out_shape=jax.ShapeDtypeStruct((N, H), jnp.float32),
        grid=(N // BN,),
        in_specs=[
            pl.BlockSpec((BN, D), lambda i: (i, 0)),
            pl.BlockSpec((BN, D), lambda i: (i, 0)),
            pl.BlockSpec((BN, D), lambda i: (i, 0)),
            pl.BlockSpec((D, H), lambda i: (0, 0)),
            pl.BlockSpec((1, H), lambda i: (0, 0)),
        ],
        out_specs=pl.BlockSpec((BN, H), lambda i: (i, 0)),
    )(h, p0, p1, w, b.reshape(1, H))


def _mm2_head_body(h_ref, p0_ref, p1_ref, w_ref, b_ref, wf_ref, bf_ref,
                   o_ref, acc_ref):
    i = pl.program_id(0)
    s = h_ref[...] + p0_ref[...] + p1_ref[...]
    y = jnp.dot(s, w_ref[...], preferred_element_type=jnp.float32) + b_ref[...]
    h2 = jnp.maximum(y, 0.0)
    colsum = jnp.sum(h2, axis=0, keepdims=True)

    @pl.when(i == 0)
    def _():
        acc_ref[...] = colsum

    @pl.when(i > 0)
    def _():
        acc_ref[...] = acc_ref[...] + colsum

    @pl.when(i == pl.num_programs(0) - 1)
    def _():
        mean = acc_ref[...] * (1.0 / N)
        o_ref[...] = (jnp.dot(mean, wf_ref[...],
                              preferred_element_type=jnp.float32) + bf_ref[...])


def _mm2_head(h, p0, p1, w, b, wf, bf):
    return pl.pallas_call(
        _mm2_head_body,
        out_shape=jax.ShapeDtypeStruct((1, C), jnp.float32),
        grid=(N // BN,),
        in_specs=[
            pl.BlockSpec((BN, D), lambda i: (i, 0)),
            pl.BlockSpec((BN, D), lambda i: (i, 0)),
            pl.BlockSpec((BN, D), lambda i: (i, 0)),
            pl.BlockSpec((D, H), lambda i: (0, 0)),
            pl.BlockSpec((1, H), lambda i: (0, 0)),
            pl.BlockSpec((H, C), lambda i: (0, 0)),
            pl.BlockSpec((1, C), lambda i: (0, 0)),
        ],
        out_specs=pl.BlockSpec((1, C), lambda i: (0, 0)),
        scratch_shapes=[pltpu.VMEM((1, H), jnp.float32)],
    )(h, p0, p1, w, b.reshape(1, H), wf, bf.reshape(1, C))


def kernel(x, edge_index, W1, b1, W2, b2, Wf, bf):
    src = edge_index[0].astype(jnp.int32).reshape(NW, SB_N, SB_C, CHUNK)
    dst = edge_index[1].astype(jnp.int32).reshape(NW, SB_N, SB_C, CHUNK)
    zeros = jnp.zeros((N_PAD, D), jnp.float32)

    p = _sc_segment_sum(x, src, dst, zeros)
    h1 = _mm_relu(x, p[:N], p[N_PAD:N_PAD + N], W1, b1)
    p2 = _sc_segment_sum(h1, src, dst, zeros)
    return _mm2_head(h1, p2[:N], p2[N_PAD:N_PAD + N], W2, b2, Wf, bf)


# speedup vs baseline: 1.0385x; 1.0385x over previous
import functools

import jax
import jax.numpy as jnp
from jax import lax
from jax.experimental import pallas as pl
from jax.experimental.pallas import tpu as pltpu
from jax.experimental.pallas import tpu_sc as plsc

N = 10000
E = 320000
D = 128
H = 128
C = 10

NC = 2
NS = 16
NW = NC * NS

EDGES_PER_W = E // NW
CHUNK = 80
CHUNKS_PER_W = EDGES_PER_W // CHUNK
SB_N = 5
SB_C = CHUNKS_PER_W // SB_N
ROWS_PER_TILE = 624
ROWS_LAST = N - 15 * ROWS_PER_TILE


def _sc_segment_sum_body(h_hbm, src_hbm, dst_hbm, zero_hbm, out_hbm,
                         src_v, dst_v, rows_a, rows_b, agg_sh,
                         sem_ga, sem_gb, sem_sa, sem_sb):
    cid = lax.axis_index("c")
    sid = lax.axis_index("s")
    wid = cid * NS + sid

    row0 = sid * ROWS_PER_TILE

    @pl.when(sid < NS - 1)
    def _():
        pltpu.sync_copy(zero_hbm.at[pl.ds(row0, ROWS_PER_TILE)],
                        agg_sh.at[pl.ds(row0, ROWS_PER_TILE)])

    @pl.when(sid == NS - 1)
    def _():
        pltpu.sync_copy(zero_hbm.at[pl.ds(row0, ROWS_LAST)],
                        agg_sh.at[pl.ds(row0, ROWS_LAST)])

    plsc.subcore_barrier()

    def _gather(i, buf, sem):
        pltpu.async_copy(h_hbm.at[src_v.at[i]], buf, sem)

    def _scatter(i, buf, sem):
        pltpu.async_copy(buf, agg_sh.at[dst_v.at[i]], sem, add=True)

    def _wait(buf, sem):
        pltpu.make_async_copy(h_hbm.at[pl.ds(0, CHUNK)], buf, sem).wait()

    def sb_body(sb, _):
        pltpu.sync_copy(src_hbm.at[wid, sb], src_v)
        pltpu.sync_copy(dst_hbm.at[wid, sb], dst_v)
        _gather(0, rows_a, sem_ga)
        _wait(rows_a, sem_ga)
        _scatter(0, rows_a, sem_sa)
        _gather(1, rows_b, sem_gb)

        def body(k, _):
            i = 2 * k + 1
            _wait(rows_b, sem_gb)
            _wait(rows_a, sem_sa)
            _scatter(i, rows_b, sem_sb)
            _gather(i + 1, rows_a, sem_ga)
            _wait(rows_a, sem_ga)
            _wait(rows_b, sem_sb)
            _scatter(i + 1, rows_a, sem_sa)
            _gather(i + 2, rows_b, sem_gb)
            return 0

        lax.fori_loop(0, (SB_C - 3) // 2, body, 0)
        _wait(rows_b, sem_gb)
        _wait(rows_a, sem_sa)
        _scatter(SB_C - 2, rows_b, sem_sb)
        _gather(SB_C - 1, rows_a, sem_ga)
        _wait(rows_a, sem_ga)
        _wait(rows_b, sem_sb)
        _scatter(SB_C - 1, rows_a, sem_sa)
        _wait(rows_a, sem_sa)
        return 0

    lax.fori_loop(0, SB_N, sb_body, 0)
    plsc.subcore_barrier()

    @pl.when(sid < NS - 1)
    def _():
        pltpu.sync_copy(agg_sh.at[pl.ds(row0, ROWS_PER_TILE)],
                        out_hbm.at[pl.ds(cid * N + row0, ROWS_PER_TILE)])

    @pl.when(sid == NS - 1)
    def _():
        pltpu.sync_copy(agg_sh.at[pl.ds(row0, ROWS_LAST)],
                        out_hbm.at[pl.ds(cid * N + row0, ROWS_LAST)])


_sc_segment_sum = functools.partial(
    pl.kernel,
    out_type=jax.ShapeDtypeStruct((NC * N, D), jnp.float32),
    mesh=plsc.VectorSubcoreMesh(core_axis_name="c", subcore_axis_name="s"),
    scratch_types=[
        pltpu.VMEM((SB_C, CHUNK), jnp.int32),
        pltpu.VMEM((SB_C, CHUNK), jnp.int32),
        pltpu.VMEM((CHUNK, D), jnp.float32),
        pltpu.VMEM((CHUNK, D), jnp.float32),
        pltpu.VMEM_SHARED((N, D), jnp.float32),
        pltpu.SemaphoreType.DMA,
        pltpu.SemaphoreType.DMA,
        pltpu.SemaphoreType.DMA,
        pltpu.SemaphoreType.DMA,
    ],
)(_sc_segment_sum_body)


BN = 2000


def _mm_relu_body(h_ref, p0_ref, p1_ref, w_ref, b_ref, o_ref):
    s = h_ref[...] + p0_ref[...] + p1_ref[...]
    y = jnp.dot(s, w_ref[...], preferred_element_type=jnp.float32) + b_ref[...]
    o_ref[...] = jnp.maximum(y, 0.0)


def _mm_relu(h, p, w, b):
    return pl.pallas_call(
        _mm_relu_body,
        out_shape=jax.ShapeDtypeStruct((N, H), jnp.float32),
        grid=(N // BN,),
        in_specs=[
            pl.BlockSpec((BN, D), lambda i: (i, 0)),
            pl.BlockSpec((BN, D), lambda i: (i, 0)),
            pl.BlockSpec((BN, D), lambda i: (N // BN + i, 0)),
            pl.BlockSpec((D, H), lambda i: (0, 0)),
            pl.BlockSpec((1, H), lambda i: (0, 0)),
        ],
        out_specs=pl.BlockSpec((BN, H), lambda i: (i, 0)),
    )(h, p, p, w, b.reshape(1, H))


def _mm2_head_body(h_ref, p0_ref, p1_ref, w_ref, b_ref, wf_ref, bf_ref,
                   o_ref, acc_ref):
    i = pl.program_id(0)
    s = h_ref[...] + p0_ref[...] + p1_ref[...]
    y = jnp.dot(s, w_ref[...], preferred_element_type=jnp.float32) + b_ref[...]
    h2 = jnp.maximum(y, 0.0)
    colsum = jnp.sum(h2, axis=0, keepdims=True)

    @pl.when(i == 0)
    def _():
        acc_ref[...] = colsum

    @pl.when(i > 0)
    def _():
        acc_ref[...] = acc_ref[...] + colsum

    @pl.when(i == pl.num_programs(0) - 1)
    def _():
        mean = acc_ref[...] * (1.0 / N)
        o_ref[...] = (jnp.dot(mean, wf_ref[...],
                              preferred_element_type=jnp.float32) + bf_ref[...])


def _mm2_head(h, p, w, b, wf, bf):
    return pl.pallas_call(
        _mm2_head_body,
        out_shape=jax.ShapeDtypeStruct((1, C), jnp.float32),
        grid=(N // BN,),
        in_specs=[
            pl.BlockSpec((BN, D), lambda i: (i, 0)),
            pl.BlockSpec((BN, D), lambda i: (i, 0)),
            pl.BlockSpec((BN, D), lambda i: (N // BN + i, 0)),
            pl.BlockSpec((D, H), lambda i: (0, 0)),
            pl.BlockSpec((1, H), lambda i: (0, 0)),
            pl.BlockSpec((H, C), lambda i: (0, 0)),
            pl.BlockSpec((1, C), lambda i: (0, 0)),
        ],
        out_specs=pl.BlockSpec((1, C), lambda i: (0, 0)),
        scratch_shapes=[pltpu.VMEM((1, H), jnp.float32)],
    )(h, p, p, w, b.reshape(1, H), wf, bf.reshape(1, C))


def kernel(x, edge_index, W1, b1, W2, b2, Wf, bf):
    src = edge_index[0].astype(jnp.int32).reshape(NW, SB_N, SB_C, CHUNK)
    dst = edge_index[1].astype(jnp.int32).reshape(NW, SB_N, SB_C, CHUNK)
    zeros = jnp.zeros((N, D), jnp.float32)

    p = _sc_segment_sum(x, src, dst, zeros)
    h1 = _mm_relu(x, p, W1, b1)
    p2 = _sc_segment_sum(h1, src, dst, zeros)
    return _mm2_head(h1, p2, W2, b2, Wf, bf)


# speedup vs baseline: 1.0790x; 1.0390x over previous
import functools

import jax
import jax.numpy as jnp
from jax import lax
from jax.experimental import pallas as pl
from jax.experimental.pallas import tpu as pltpu
from jax.experimental.pallas import tpu_sc as plsc

N = 10000
E = 320000
D = 128
H = 128
C = 10

NC = 2
NS = 16
NW = NC * NS

EDGES_PER_W = E // NW
CHUNK = 80
CHUNKS_PER_W = EDGES_PER_W // CHUNK
SB_N = 5
SB_C = CHUNKS_PER_W // SB_N
ROWS_PER_TILE = 624
ROWS_LAST = N - 15 * ROWS_PER_TILE


def _sc_segment_sum_body(h_hbm, src_hbm, dst_hbm, zero_hbm, out_hbm,
                         src_a, dst_a, src_b, dst_b, rows_a, rows_b, agg_sh,
                         sem_i, sem_ga, sem_gb, sem_sa, sem_sb):
    cid = lax.axis_index("c")
    sid = lax.axis_index("s")
    wid = cid * NS + sid
    idx_bufs = ((src_a, dst_a), (src_b, dst_b))

    def _idx_load(sb, bufs):
        pltpu.async_copy(src_hbm.at[wid, sb], bufs[0], sem_i)
        pltpu.async_copy(dst_hbm.at[wid, sb], bufs[1], sem_i)

    def _idx_wait(bufs):
        pltpu.make_async_copy(src_hbm.at[wid, 0], bufs[0], sem_i).wait()
        pltpu.make_async_copy(dst_hbm.at[wid, 0], bufs[1], sem_i).wait()

    def _gather(sv, i, buf, sem):
        pltpu.async_copy(h_hbm.at[sv.at[i]], buf, sem)

    def _scatter(dv, i, buf, sem):
        pltpu.async_copy(buf, agg_sh.at[dv.at[i]], sem, add=True)

    def _wait(buf, sem):
        pltpu.make_async_copy(h_hbm.at[pl.ds(0, CHUNK)], buf, sem).wait()

    _idx_load(0, idx_bufs[0])
    row0 = sid * ROWS_PER_TILE

    @pl.when(sid < NS - 1)
    def _():
        pltpu.sync_copy(zero_hbm.at[pl.ds(row0, ROWS_PER_TILE)],
                        agg_sh.at[pl.ds(row0, ROWS_PER_TILE)])

    @pl.when(sid == NS - 1)
    def _():
        pltpu.sync_copy(zero_hbm.at[pl.ds(row0, ROWS_LAST)],
                        agg_sh.at[pl.ds(row0, ROWS_LAST)])

    _idx_wait(idx_bufs[0])
    _gather(src_a, 0, rows_a, sem_ga)
    _gather(src_a, 1, rows_b, sem_gb)
    plsc.subcore_barrier()

    for sb in range(SB_N):
        sv, dv = idx_bufs[sb % 2]
        nsv_ndv = idx_bufs[(sb + 1) % 2]
        if sb + 1 < SB_N:
            _idx_load(sb + 1, nsv_ndv)

        _wait(rows_a, sem_ga)
        _scatter(dv, 0, rows_a, sem_sa)

        def body(k, _, sv=sv, dv=dv):
            i = 2 * k + 1
            _wait(rows_b, sem_gb)
            _wait(rows_a, sem_sa)
            _scatter(dv, i, rows_b, sem_sb)
            _gather(sv, i + 1, rows_a, sem_ga)
            _wait(rows_a, sem_ga)
            _wait(rows_b, sem_sb)
            _scatter(dv, i + 1, rows_a, sem_sa)
            _gather(sv, i + 2, rows_b, sem_gb)
            return 0

        lax.fori_loop(0, (SB_C - 3) // 2, body, 0)
        _wait(rows_b, sem_gb)
        _wait(rows_a, sem_sa)
        _scatter(dv, SB_C - 2, rows_b, sem_sb)
        _gather(sv, SB_C - 1, rows_a, sem_ga)
        _wait(rows_a, sem_ga)
        _wait(rows_b, sem_sb)
        _scatter(dv, SB_C - 1, rows_a, sem_sa)
        _wait(rows_a, sem_sa)
        if sb + 1 < SB_N:
            _idx_wait(nsv_ndv)
            _gather(nsv_ndv[0], 0, rows_a, sem_ga)
            _gather(nsv_ndv[0], 1, rows_b, sem_gb)

    plsc.subcore_barrier()

    @pl.when(sid < NS - 1)
    def _():
        pltpu.sync_copy(agg_sh.at[pl.ds(row0, ROWS_PER_TILE)],
                        out_hbm.at[pl.ds(cid * N + row0, ROWS_PER_TILE)])

    @pl.when(sid == NS - 1)
    def _():
        pltpu.sync_copy(agg_sh.at[pl.ds(row0, ROWS_LAST)],
                        out_hbm.at[pl.ds(cid * N + row0, ROWS_LAST)])


_sc_segment_sum = functools.partial(
    pl.kernel,
    out_type=jax.ShapeDtypeStruct((NC * N, D), jnp.float32),
    mesh=plsc.VectorSubcoreMesh(core_axis_name="c", subcore_axis_name="s"),
    scratch_types=[
        pltpu.VMEM((SB_C, CHUNK), jnp.int32),
        pltpu.VMEM((SB_C, CHUNK), jnp.int32),
        pltpu.VMEM((SB_C, CHUNK), jnp.int32),
        pltpu.VMEM((SB_C, CHUNK), jnp.int32),
        pltpu.VMEM((CHUNK, D), jnp.float32),
        pltpu.VMEM((CHUNK, D), jnp.float32),
        pltpu.VMEM_SHARED((N, D), jnp.float32),
        pltpu.SemaphoreType.DMA,
        pltpu.SemaphoreType.DMA,
        pltpu.SemaphoreType.DMA,
        pltpu.SemaphoreType.DMA,
        pltpu.SemaphoreType.DMA,
    ],
)(_sc_segment_sum_body)


BN = 2000


def _mm_relu_body(h_ref, p0_ref, p1_ref, w_ref, b_ref, o_ref):
    s = h_ref[...] + p0_ref[...] + p1_ref[...]
    y = jnp.dot(s, w_ref[...], preferred_element_type=jnp.float32) + b_ref[...]
    o_ref[...] = jnp.maximum(y, 0.0)


def _mm_relu(h, p, w, b):
    return pl.pallas_call(
        _mm_relu_body,
        out_shape=jax.ShapeDtypeStruct((N, H), jnp.float32),
        grid=(N // BN,),
        in_specs=[
            pl.BlockSpec((BN, D), lambda i: (i, 0)),
            pl.BlockSpec((BN, D), lambda i: (i, 0)),
            pl.BlockSpec((BN, D), lambda i: (N // BN + i, 0)),
            pl.BlockSpec((D, H), lambda i: (0, 0)),
            pl.BlockSpec((1, H), lambda i: (0, 0)),
        ],
        out_specs=pl.BlockSpec((BN, H), lambda i: (i, 0)),
    )(h, p, p, w, b.reshape(1, H))


def _mm2_head_body(h_ref, p0_ref, p1_ref, w_ref, b_ref, wf_ref, bf_ref,
                   o_ref, acc_ref):
    i = pl.program_id(0)
    s = h_ref[...] + p0_ref[...] + p1_ref[...]
    y = jnp.dot(s, w_ref[...], preferred_element_type=jnp.float32) + b_ref[...]
    h2 = jnp.maximum(y, 0.0)
    colsum = jnp.sum(h2, axis=0, keepdims=True)

    @pl.when(i == 0)
    def _():
        acc_ref[...] = colsum

    @pl.when(i > 0)
    def _():
        acc_ref[...] = acc_ref[...] + colsum

    @pl.when(i == pl.num_programs(0) - 1)
    def _():
        mean = acc_ref[...] * (1.0 / N)
        o_ref[...] = (jnp.dot(mean, wf_ref[...],
                              preferred_element_type=jnp.float32) + bf_ref[...])


def _mm2_head(h, p, w, b, wf, bf):
    return pl.pallas_call(
        _mm2_head_body,
        out_shape=jax.ShapeDtypeStruct((1, C), jnp.float32),
        grid=(N // BN,),
        in_specs=[
            pl.BlockSpec((BN, D), lambda i: (i, 0)),
            pl.BlockSpec((BN, D), lambda i: (i, 0)),
            pl.BlockSpec((BN, D), lambda i: (N // BN + i, 0)),
            pl.BlockSpec((D, H), lambda i: (0, 0)),
            pl.BlockSpec((1, H), lambda i: (0, 0)),
            pl.BlockSpec((H, C), lambda i: (0, 0)),
            pl.BlockSpec((1, C), lambda i: (0, 0)),
        ],
        out_specs=pl.BlockSpec((1, C), lambda i: (0, 0)),
        scratch_shapes=[pltpu.VMEM((1, H), jnp.float32)],
    )(h, p, p, w, b.reshape(1, H), wf, bf.reshape(1, C))


def kernel(x, edge_index, W1, b1, W2, b2, Wf, bf):
    src = edge_index[0].astype(jnp.int32).reshape(NW, SB_N, SB_C, CHUNK)
    dst = edge_index[1].astype(jnp.int32).reshape(NW, SB_N, SB_C, CHUNK)
    zeros = jnp.zeros((N, D), jnp.float32)

    p = _sc_segment_sum(x, src, dst, zeros)
    h1 = _mm_relu(x, p, W1, b1)
    p2 = _sc_segment_sum(h1, src, dst, zeros)
    return _mm2_head(h1, p2, W2, b2, Wf, bf)


# speedup vs baseline: 1.0837x; 1.0044x over previous
import functools

import jax
import jax.numpy as jnp
from jax import lax
from jax.experimental import pallas as pl
from jax.experimental.pallas import tpu as pltpu
from jax.experimental.pallas import tpu_sc as plsc

N = 10000
E = 320000
D = 128
H = 128
C = 10

NC = 2
NS = 16
NW = NC * NS

EDGES_PER_W = E // NW
CHUNK = 80
CHUNKS_PER_W = EDGES_PER_W // CHUNK
SB_N = 5
SB_C = CHUNKS_PER_W // SB_N
ROWS_PER_TILE = 624
ROWS_LAST = N - 15 * ROWS_PER_TILE


def _sc_segment_sum_body(h_hbm, src_hbm, dst_hbm, zero_hbm, out_hbm,
                         src_a, dst_a, src_b, dst_b, rows_a, rows_b, agg_sh,
                         sem_i, sem_ga, sem_gb, sem_sa, sem_sb):
    cid = lax.axis_index("c")
    sid = lax.axis_index("s")
    wid = cid * NS + sid
    idx_bufs = ((src_a, dst_a), (src_b, dst_b))

    def _idx_load(sb, bufs):
        pltpu.async_copy(src_hbm.at[wid, sb], bufs[0], sem_i)
        pltpu.async_copy(dst_hbm.at[wid, sb], bufs[1], sem_i)

    def _idx_wait(bufs):
        pltpu.make_async_copy(src_hbm.at[wid, 0], bufs[0], sem_i).wait()
        pltpu.make_async_copy(dst_hbm.at[wid, 0], bufs[1], sem_i).wait()

    def _gather(sv, i, buf, sem):
        pltpu.async_copy(h_hbm.at[sv.at[i]], buf, sem)

    def _scatter(dv, i, buf, sem):
        pltpu.async_copy(buf, agg_sh.at[dv.at[i]], sem, add=True)

    def _wait(buf, sem):
        pltpu.make_async_copy(h_hbm.at[pl.ds(0, CHUNK)], buf, sem).wait()

    _idx_load(0, idx_bufs[0])
    row0 = sid * ROWS_PER_TILE
    init_hbm = (h_hbm, zero_hbm)

    for c in range(NC):
        @pl.when(jnp.logical_and(cid == c, sid < NS - 1))
        def _(c=c):
            pltpu.sync_copy(init_hbm[c].at[pl.ds(row0, ROWS_PER_TILE)],
                            agg_sh.at[pl.ds(row0, ROWS_PER_TILE)])

        @pl.when(jnp.logical_and(cid == c, sid == NS - 1))
        def _(c=c):
            pltpu.sync_copy(init_hbm[c].at[pl.ds(row0, ROWS_LAST)],
                            agg_sh.at[pl.ds(row0, ROWS_LAST)])

    _idx_wait(idx_bufs[0])
    _gather(src_a, 0, rows_a, sem_ga)
    _gather(src_a, 1, rows_b, sem_gb)
    plsc.subcore_barrier()

    for sb in range(SB_N):
        sv, dv = idx_bufs[sb % 2]
        nsv_ndv = idx_bufs[(sb + 1) % 2]
        if sb + 1 < SB_N:
            _idx_load(sb + 1, nsv_ndv)

        _wait(rows_a, sem_ga)
        _scatter(dv, 0, rows_a, sem_sa)

        def body(k, _, sv=sv, dv=dv):
            i = 2 * k + 1
            _wait(rows_b, sem_gb)
            _wait(rows_a, sem_sa)
            _scatter(dv, i, rows_b, sem_sb)
            _gather(sv, i + 1, rows_a, sem_ga)
            _wait(rows_a, sem_ga)
            _wait(rows_b, sem_sb)
            _scatter(dv, i + 1, rows_a, sem_sa)
            _gather(sv, i + 2, rows_b, sem_gb)
            return 0

        lax.fori_loop(0, (SB_C - 3) // 2, body, 0)
        _wait(rows_b, sem_gb)
        _wait(rows_a, sem_sa)
        _scatter(dv, SB_C - 2, rows_b, sem_sb)
        _gather(sv, SB_C - 1, rows_a, sem_ga)
        _wait(rows_a, sem_ga)
        _wait(rows_b, sem_sb)
        _scatter(dv, SB_C - 1, rows_a, sem_sa)
        _wait(rows_a, sem_sa)
        if sb + 1 < SB_N:
            _idx_wait(nsv_ndv)
            _gather(nsv_ndv[0], 0, rows_a, sem_ga)
            _gather(nsv_ndv[0], 1, rows_b, sem_gb)

    plsc.subcore_barrier()

    @pl.when(sid < NS - 1)
    def _():
        pltpu.sync_copy(agg_sh.at[pl.ds(row0, ROWS_PER_TILE)],
                        out_hbm.at[pl.ds(cid * N + row0, ROWS_PER_TILE)])

    @pl.when(sid == NS - 1)
    def _():
        pltpu.sync_copy(agg_sh.at[pl.ds(row0, ROWS_LAST)],
                        out_hbm.at[pl.ds(cid * N + row0, ROWS_LAST)])


_sc_segment_sum = functools.partial(
    pl.kernel,
    out_type=jax.ShapeDtypeStruct((NC * N, D), jnp.float32),
    mesh=plsc.VectorSubcoreMesh(core_axis_name="c", subcore_axis_name="s"),
    scratch_types=[
        pltpu.VMEM((SB_C, CHUNK), jnp.int32),
        pltpu.VMEM((SB_C, CHUNK), jnp.int32),
        pltpu.VMEM((SB_C, CHUNK), jnp.int32),
        pltpu.VMEM((SB_C, CHUNK), jnp.int32),
        pltpu.VMEM((CHUNK, D), jnp.float32),
        pltpu.VMEM((CHUNK, D), jnp.float32),
        pltpu.VMEM_SHARED((N, D), jnp.float32),
        pltpu.SemaphoreType.DMA,
        pltpu.SemaphoreType.DMA,
        pltpu.SemaphoreType.DMA,
        pltpu.SemaphoreType.DMA,
        pltpu.SemaphoreType.DMA,
    ],
)(_sc_segment_sum_body)


BN = 2000


def _mm_relu_body(p0_ref, p1_ref, w_ref, b_ref, o_ref):
    s = p0_ref[...] + p1_ref[...]
    y = jnp.dot(s, w_ref[...], preferred_element_type=jnp.float32) + b_ref[...]
    o_ref[...] = jnp.maximum(y, 0.0)


def _mm_relu(p, w, b):
    return pl.pallas_call(
        _mm_relu_body,
        out_shape=jax.ShapeDtypeStruct((N, H), jnp.float32),
        grid=(N // BN,),
        in_specs=[
            pl.BlockSpec((BN, D), lambda i: (i, 0)),
            pl.BlockSpec((BN, D), lambda i: (N // BN + i, 0)),
            pl.BlockSpec((D, H), lambda i: (0, 0)),
            pl.BlockSpec((1, H), lambda i: (0, 0)),
        ],
        out_specs=pl.BlockSpec((BN, H), lambda i: (i, 0)),
    )(p, p, w, b.reshape(1, H))


def _mm2_head_body(p0_ref, p1_ref, w_ref, b_ref, wf_ref, bf_ref,
                   o_ref, acc_ref):
    i = pl.program_id(0)
    s = p0_ref[...] + p1_ref[...]
    y = jnp.dot(s, w_ref[...], preferred_element_type=jnp.float32) + b_ref[...]
    h2 = jnp.maximum(y, 0.0)
    colsum = jnp.sum(h2, axis=0, keepdims=True)

    @pl.when(i == 0)
    def _():
        acc_ref[...] = colsum

    @pl.when(i > 0)
    def _():
        acc_ref[...] = acc_ref[...] + colsum

    @pl.when(i == pl.num_programs(0) - 1)
    def _():
        mean = acc_ref[...] * (1.0 / N)
        o_ref[...] = (jnp.dot(mean, wf_ref[...],
                              preferred_element_type=jnp.float32) + bf_ref[...])


def _mm2_head(p, w, b, wf, bf):
    return pl.pallas_call(
        _mm2_head_body,
        out_shape=jax.ShapeDtypeStruct((1, C), jnp.float32),
        grid=(N // BN,),
        in_specs=[
            pl.BlockSpec((BN, D), lambda i: (i, 0)),
            pl.BlockSpec((BN, D), lambda i: (N // BN + i, 0)),
            pl.BlockSpec((D, H), lambda i: (0, 0)),
            pl.BlockSpec((1, H), lambda i: (0, 0)),
            pl.BlockSpec((H, C), lambda i: (0, 0)),
            pl.BlockSpec((1, C), lambda i: (0, 0)),
        ],
        out_specs=pl.BlockSpec((1, C), lambda i: (0, 0)),
        scratch_shapes=[pltpu.VMEM((1, H), jnp.float32)],
    )(p, p, w, b.reshape(1, H), wf, bf.reshape(1, C))


def kernel(x, edge_index, W1, b1, W2, b2, Wf, bf):
    src = edge_index[0].astype(jnp.int32).reshape(NW, SB_N, SB_C, CHUNK)
    dst = edge_index[1].astype(jnp.int32).reshape(NW, SB_N, SB_C, CHUNK)
    zeros = jnp.zeros((N, D), jnp.float32)

    p = _sc_segment_sum(x, src, dst, zeros)
    h1 = _mm_relu(p, W1, b1)
    p2 = _sc_segment_sum(h1, src, dst, zeros)
    return _mm2_head(p2, W2, b2, Wf, bf)
